# baseline (speedup 1.0000x reference)
import functools

import jax
import jax.numpy as jnp
from jax import lax
from jax.experimental import pallas as pl
from jax.experimental.pallas import tpu as pltpu
from jax.experimental.pallas import tpu_sc as plsc

BATCH = 16384
NROWS = 1000000
DIM = 32
LANES = 16
NUM_CORES = 2
NUM_SUBCORES = 16
NW = NUM_CORES * NUM_SUBCORES
BPW = BATCH // NW
GROUPS = BPW // LANES

RT_TOTAL = NROWS // 128
TAIL_LO = RT_TOTAL * 128
NTAIL = NROWS - TAIL_LO
TPW = (RT_TOTAL + NW - 1) // NW
CROWS = 512
NCHUNK2 = (TPW * 128 + 2 * CROWS - 1) // (2 * CROWS)
FMAX = (RT_TOTAL - 4) * 128
LCAP = BATCH + LANES
STG = 2048
ROWB = LANES * 128 * 4
CHB = 8 * CROWS * 4 * 4

_MESH = plsc.VectorSubcoreMesh(core_axis_name="c", subcore_axis_name="s")

_CPARAMS = pltpu.CompilerParams(needs_layout_passes=False)
_CPARAMS_LIN = pltpu.CompilerParams(
    needs_layout_passes=False, use_tc_tiling_on_sc=False)


def _rsqrt_newton(x):
    i = plsc.bitcast(x, jnp.int32)
    i = jnp.int32(0x5F3759DF) - (i >> 1)
    y = plsc.bitcast(i, jnp.float32)
    half_x = 0.5 * x
    for _ in range(3):
        y = y * (1.5 - half_x * y * y)
    return y


@functools.partial(
    pl.kernel,
    mesh=_MESH,
    compiler_params=_CPARAMS,
    out_type=(
        jax.ShapeDtypeStruct((LCAP, 128), jnp.float32),
        jax.ShapeDtypeStruct((LCAP, 128), jnp.float32),
    ),
    scratch_types=[
        pltpu.VMEM((STG,), jnp.int32),
        pltpu.VMEM((LCAP,), jnp.int32),
        pltpu.VMEM((LCAP,), jnp.int32),
        pltpu.VMEM((LCAP,), jnp.int32),
        pltpu.VMEM((LCAP,), jnp.int32),
        pltpu.VMEM((DIM, CROWS), jnp.float32),
        pltpu.VMEM((DIM, CROWS), jnp.float32),
        pltpu.VMEM((NTAIL, DIM), jnp.float32),
        pltpu.VMEM((NTAIL, DIM), jnp.float32),
        pltpu.VMEM((4 * LANES, 128), jnp.float32),
        pltpu.SemaphoreType.DMA,
        pltpu.SemaphoreType.DMA,
    ],
)
def _extract_sc(u_hbm, v_hbm, usersQ, itemsQ, tail_u, tail_v,
                ue_hbm, ve_hbm,
                stg, ulist, upos, vlist, vpos, bufa, bufb, tbu, tbv,
                oring, ssem, wsem):
    wid = lax.axis_index("s") * NUM_CORES + lax.axis_index("c")
    rt_lo = jnp.minimum(wid * TPW, RT_TOTAL)
    rt_hi = jnp.minimum(rt_lo + TPW, RT_TOTAL)
    row_lo = rt_lo * 128
    row_hi = jnp.where(rt_hi == RT_TOTAL, NROWS, rt_hi * 128)
    iota = lax.iota(jnp.int32, LANES)

    pltpu.sync_copy(tail_u, tbu)
    pltpu.sync_copy(tail_v, tbv)

    def build(idx_hbm, lref, pref):
        def stage_body(sb, cnt):
            pltpu.sync_copy(idx_hbm.at[pl.ds(sb * STG, STG)], stg)

            def vec_body(k, cnt):
                ii = stg[pl.ds(k * LANES, LANES)]
                m = (ii >= row_lo) & (ii < row_hi)
                plsc.store_compressed(lref.at[pl.ds(cnt, LANES)], ii, mask=m)
                pos = sb * STG + k * LANES + iota
                plsc.store_compressed(pref.at[pl.ds(cnt, LANES)], pos, mask=m)
                npc = plsc.all_reduce_population_count(m)
                return cnt + jnp.max(npc)

            return lax.fori_loop(0, STG // LANES, vec_body, cnt)

        return lax.fori_loop(0, BATCH // STG, stage_body, jnp.int32(0))

    ucnt = build(u_hbm, ulist, upos)
    vcnt = build(v_hbm, vlist, vpos)

    def fetch(tabQ, buf, c):
        base = pl.multiple_of(
            jnp.minimum(row_lo + c * CROWS, FMAX), 128)
        for c8 in range(4):
            pltpu.async_copy(
                tabQ.at[c8, :, pl.ds(base, CROWS)],
                buf.at[pl.ds(c8 * 8, 8)], ssem)

    def wait_chunk():
        for _ in range(4):
            pltpu.make_async_copy(
                usersQ.at[0, :, pl.ds(0, CROWS)],
                bufa.at[pl.ds(0, 8)], ssem).wait()

    def wait_one_scatter():
        pltpu.make_async_copy(
            ue_hbm.at[pl.ds(0, LANES)],
            oring.at[pl.ds(0, LANES)], wsem).wait()

    def run_table(tabQ, lref, pref, cnt, out_hbm, tb, s0):
        kmax = (cnt + LANES - 1) // LANES

        def process(buf, lo, hi, fbase, s, dim_major):
            def entry_body(k, s):
                lv = lref[pl.ds(k * LANES, LANES)]
                pv = pref[pl.ds(k * LANES, LANES)]
                valid = (k * LANES + iota) < cnt
                m = valid & (lv >= lo) & (lv < hi)
                nhit = jnp.max(plsc.all_reduce_population_count(m))

                def do_extract(s):
                    def wait_one(x):
                        wait_one_scatter()
                        return x

                    s = lax.cond(s >= 4, wait_one, lambda x: x, s)
                    slot = s & 3
                    off = jnp.where(m, lv - fbase, 0)
                    orow = slot * LANES + iota
                    for d in range(DIM):
                        d_v = jnp.full((LANES,), d, jnp.int32)
                        if dim_major:
                            vals = plsc.load_gather(buf, [d_v, off], mask=m)
                        else:
                            vals = plsc.load_gather(buf, [off, d_v], mask=m)
                        plsc.store_scatter(oring, [orow, d_v], vals, mask=m)
                    pos_safe = jnp.where(m, pv, BATCH + iota)
                    pltpu.async_copy(
                        oring.at[pl.ds(slot * LANES, LANES)],
                        out_hbm.at[pos_safe], wsem)
                    return s + 1

                return lax.cond(nhit > 0, do_extract, lambda s: s, s)

            return lax.fori_loop(0, kmax, entry_body, s)

        fetch(tabQ, bufa, jnp.int32(0))

        def chunk_bounds(c):
            lo = row_lo + c * CROWS
            hi = jnp.minimum(lo + CROWS, jnp.minimum(row_hi, TAIL_LO))
            fbase = jnp.minimum(lo, FMAX)
            return lo, hi, fbase

        def pair_body(t, s):
            wait_chunk()
            fetch(tabQ, bufb, 2 * t + 1)
            lo, hi, fb = chunk_bounds(2 * t)
            s = process(bufa, lo, hi, fb, s, True)
            wait_chunk()
            fetch(tabQ, bufa, 2 * t + 2)
            lo, hi, fb = chunk_bounds(2 * t + 1)
            return process(bufb, lo, hi, fb, s, True)

        s = lax.fori_loop(0, NCHUNK2, pair_body, s0)
        wait_chunk()

        s = process(tb, jnp.int32(TAIL_LO),
                    jnp.minimum(row_hi, NROWS), jnp.int32(TAIL_LO), s, False)
        return s

    s = run_table(usersQ, ulist, upos, ucnt, ue_hbm, tbu, jnp.int32(0))
    s = run_table(itemsQ, vlist, vpos, vcnt, ve_hbm, tbv, s)

    def drain_body(_, x):
        wait_one_scatter()
        return x

    lax.fori_loop(0, jnp.minimum(s, 4), drain_body, 0)


HALF = BPW // 2


@functools.partial(
    pl.kernel,
    mesh=_MESH,
    compiler_params=_CPARAMS_LIN,
    out_type=jax.ShapeDtypeStruct((BATCH,), jnp.float32),
    scratch_types=[
        pltpu.VMEM((HALF, 128), jnp.float32),
        pltpu.VMEM((HALF, 128), jnp.float32),
        pltpu.VMEM((BPW,), jnp.float32),
    ],
)
def _score_sc(ue_hbm, ve_hbm, out_hbm, urows, vrows, out_v):
    wid = lax.axis_index("s") * NUM_CORES + lax.axis_index("c")
    base = wid * BPW
    iota = lax.iota(jnp.int32, LANES)

    for h in range(2):
        pltpu.sync_copy(ue_hbm.at[pl.ds(base + h * HALF, HALF)], urows)
        pltpu.sync_copy(ve_hbm.at[pl.ds(base + h * HALF, HALF)], vrows)

        def group_body(g, _):
            rows_v = g * LANES + iota
            nu = jnp.zeros((LANES,), jnp.float32)
            nv = jnp.zeros((LANES,), jnp.float32)
            dot = jnp.zeros((LANES,), jnp.float32)
            for d in range(DIM):
                d_v = jnp.full((LANES,), d, jnp.int32)
                ud = plsc.load_gather(urows, [rows_v, d_v])
                vd = plsc.load_gather(vrows, [rows_v, d_v])
                nu = nu + ud * ud
                nv = nv + vd * vd
                dot = dot + ud * vd
            su = jnp.minimum(1.0, _rsqrt_newton(nu))
            sv = jnp.minimum(1.0, _rsqrt_newton(nv))
            x = dot * su * sv
            out_v[pl.ds(h * HALF + g * LANES, LANES)] = (
                1.0 / (1.0 + jnp.exp(-x)))
            return 0

        lax.fori_loop(0, HALF // LANES, group_body, 0)

    pltpu.sync_copy(out_v, out_hbm.at[pl.ds(base, BPW)])


def kernel(u, v, users, items):
    usersQ = users.T.reshape(4, 8, NROWS)
    itemsQ = items.T.reshape(4, 8, NROWS)
    ue, ve = _extract_sc(u, v, usersQ, itemsQ,
                         users[TAIL_LO:], items[TAIL_LO:])
    return _score_sc(ue, ve)


# speedup vs baseline: 1.7037x; 1.7037x over previous
import functools

import jax
import jax.numpy as jnp
from jax import lax
from jax.experimental import pallas as pl
from jax.experimental.pallas import tpu as pltpu
from jax.experimental.pallas import tpu_sc as plsc

BATCH = 16384
DIM = 32
LANES = 16
NUM_CORES = 2
NUM_SUBCORES = 16
NW = NUM_CORES * NUM_SUBCORES
BPW = BATCH // NW
CHUNK = 128
NCHUNK = BPW // CHUNK
GROUPS = BPW // LANES

_MESH = plsc.VectorSubcoreMesh(core_axis_name="c", subcore_axis_name="s")


def _rsqrt_newton(x):
    i = plsc.bitcast(x, jnp.int32)
    i = jnp.int32(0x5F3759DF) - (i >> 1)
    y = plsc.bitcast(i, jnp.float32)
    half_x = 0.5 * x
    for _ in range(3):
        y = y * (1.5 - half_x * y * y)
    return y


@functools.partial(
    pl.kernel,
    mesh=_MESH,
    compiler_params=pltpu.CompilerParams(
        needs_layout_passes=False, use_tc_tiling_on_sc=False),
    out_type=jax.ShapeDtypeStruct((BATCH,), jnp.float32),
    scratch_types=[
        pltpu.VMEM((NCHUNK, CHUNK), jnp.int32),
        pltpu.VMEM((NCHUNK, CHUNK), jnp.int32),
        pltpu.VMEM((BPW, DIM), jnp.float32),
        pltpu.VMEM((BPW, DIM), jnp.float32),
        pltpu.VMEM((BPW,), jnp.float32),
        pltpu.SemaphoreType.DMA,
    ],
)
def _als_sc(u_hbm, v_hbm, users_hbm, items_hbm, out_hbm,
            uidx, vidx, urows, vrows, out_v, sem):
    wid = lax.axis_index("s") * NUM_CORES + lax.axis_index("c")
    base = wid * BPW

    for j in range(NCHUNK):
        pltpu.sync_copy(u_hbm.at[pl.ds(base + j * CHUNK, CHUNK)], uidx.at[j])
        pltpu.sync_copy(v_hbm.at[pl.ds(base + j * CHUNK, CHUNK)], vidx.at[j])

    copies = []
    for j in range(NCHUNK):
        copies.append(pltpu.async_copy(
            users_hbm.at[uidx.at[j]], urows.at[pl.ds(j * CHUNK, CHUNK)], sem))
        copies.append(pltpu.async_copy(
            items_hbm.at[vidx.at[j]], vrows.at[pl.ds(j * CHUNK, CHUNK)], sem))
    for c in copies:
        c.wait()

    iota = lax.iota(jnp.int32, LANES)

    def group_body(g, _):
        rows_v = g * LANES + iota
        nu = jnp.zeros((LANES,), jnp.float32)
        nv = jnp.zeros((LANES,), jnp.float32)
        dot = jnp.zeros((LANES,), jnp.float32)
        for d in range(DIM):
            d_v = jnp.full((LANES,), d, jnp.int32)
            ud = plsc.load_gather(urows, [rows_v, d_v])
            vd = plsc.load_gather(vrows, [rows_v, d_v])
            nu = nu + ud * ud
            nv = nv + vd * vd
            dot = dot + ud * vd
        su = jnp.minimum(1.0, _rsqrt_newton(nu))
        sv = jnp.minimum(1.0, _rsqrt_newton(nv))
        x = dot * su * sv
        logit = 1.0 / (1.0 + jnp.exp(-x))
        out_v[pl.ds(g * LANES, LANES)] = logit
        return 0

    lax.fori_loop(0, GROUPS, group_body, 0)

    pltpu.sync_copy(out_v, out_hbm.at[pl.ds(base, BPW)])


def kernel(u, v, users, items):
    users_r = users.astype(jnp.bfloat16).astype(jnp.float32)
    items_r = items.astype(jnp.bfloat16).astype(jnp.float32)
    return _als_sc(u, v, users_r, items_r)
